# Initial kernel scaffold; baseline (speedup 1.0000x reference)
import jax
import jax.numpy as jnp
from jax.experimental import pallas as pl


def kernel(x, w_ih, Wq_idx, bq_idx, Wk_idx, bk_idx, Wqkv, bqkv, Wout, bout):
    raise NotImplementedError("write your pallas kernel here")


# speedup vs baseline: 9.8877x; 9.8877x over previous
import functools

import jax
import jax.numpy as jnp
from jax.experimental import pallas as pl
from jax.experimental.pallas import tpu as pltpu

NEG = -1e30
TOPK = 256
BLK_T = 256


def _monotone_i32(x_f32):
    b = jax.lax.bitcast_convert_type(x_f32, jnp.int32)
    mask = jax.lax.shift_right_arithmetic(b, 31) & jnp.int32(0x7FFFFFFF)
    return b ^ mask


def _proj_kernel(x_ref, w_ref, b_ref, o_ref):
    o_ref[...] = (
        jnp.dot(x_ref[...], w_ref[...], preferred_element_type=jnp.float32)
        + b_ref[...]
    )


def _proj_call(x2d, wcat, bcat):
    T, D = x2d.shape
    N = wcat.shape[1]
    grid = (T // BLK_T,)
    return pl.pallas_call(
        _proj_kernel,
        grid=grid,
        in_specs=[
            pl.BlockSpec((BLK_T, D), lambda i: (i, 0)),
            pl.BlockSpec((D, N), lambda i: (0, 0)),
            pl.BlockSpec((1, N), lambda i: (0, 0)),
        ],
        out_specs=pl.BlockSpec((BLK_T, N), lambda i: (i, 0)),
        out_shape=jax.ShapeDtypeStruct((T, N), jnp.float32),
    )(x2d, wcat, bcat)


def _scores_kernel(qi_ref, ki_ref, w_ref, i_ref, thr_ref, m_ref, *, hi, di, topk):
    i = pl.program_id(0)
    T = ki_ref.shape[0]
    acc = jnp.zeros((BLK_T, T), jnp.float32)
    for h in range(hi):
        qh = qi_ref[:, h * di:(h + 1) * di]
        kh = ki_ref[:, h * di:(h + 1) * di]
        sh = jax.lax.dot_general(
            qh, kh, (((1,), (1,)), ((), ())),
            preferred_element_type=jnp.float32)
        acc = acc + w_ref[0, h] * jnp.maximum(sh, 0.0)
    t_glob = i * BLK_T + jax.lax.broadcasted_iota(jnp.int32, (BLK_T, T), 0)
    s_idx = jax.lax.broadcasted_iota(jnp.int32, (BLK_T, T), 1)
    scores = jnp.where(s_idx <= t_glob, acc, NEG)
    i_ref[...] = scores

    s = _monotone_i32(scores)
    k0 = jnp.int32(topk)
    nonneg = (s >= 0).astype(jnp.int32)
    cnt = jnp.sum(nonneg, axis=1, keepdims=True)
    cond = cnt >= k0
    c = jnp.where(cond, nonneg, 1 - nonneg)
    kk = jnp.where(cond, k0, k0 - cnt)
    thr = jnp.where(cond, jnp.int32(0), jnp.int32(-2147483648))

    def body(it, carry):
        thr, c, kk = carry
        b = 30 - it
        bitval = jax.lax.shift_left(jnp.int32(1), b)
        bit = ((s & bitval) != 0).astype(jnp.int32)
        hasbit = bit * c
        cnt1 = jnp.sum(hasbit, axis=1, keepdims=True)
        cond = cnt1 >= kk
        thr = jnp.where(cond, thr | bitval, thr)
        c = jnp.where(cond, hasbit, c * (1 - bit))
        kk = jnp.where(cond, kk, kk - cnt1)
        return thr, c, kk

    thr, c, kk = jax.lax.fori_loop(0, 31, body, (thr, c, kk))
    cnt_gt = jnp.sum((s > thr).astype(jnp.int32), axis=1, keepdims=True)
    m = k0 - cnt_gt
    thr_ref[...] = jnp.broadcast_to(thr, (BLK_T, 128))
    m_ref[...] = jnp.broadcast_to(m, (BLK_T, 128))


def _scores_call(qi, ki, w2d, hi, di):
    T = qi.shape[0]
    grid = (T // BLK_T,)
    kern = functools.partial(_scores_kernel, hi=hi, di=di, topk=TOPK)
    return pl.pallas_call(
        kern,
        grid=grid,
        in_specs=[
            pl.BlockSpec((BLK_T, hi * di), lambda i: (i, 0)),
            pl.BlockSpec((T, hi * di), lambda i: (0, 0)),
            pl.BlockSpec(memory_space=pltpu.SMEM),
        ],
        out_specs=[
            pl.BlockSpec((BLK_T, T), lambda i: (i, 0)),
            pl.BlockSpec((BLK_T, 128), lambda i: (i, 0)),
            pl.BlockSpec((BLK_T, 128), lambda i: (i, 0)),
        ],
        out_shape=[
            jax.ShapeDtypeStruct((T, T), jnp.float32),
            jax.ShapeDtypeStruct((T, 128), jnp.int32),
            jax.ShapeDtypeStruct((T, 128), jnp.int32),
        ],
    )(qi, ki, w2d)


def _attn_kernel(i_ref, thr_ref, m_ref, q_ref, k_ref, v_ref, wo_ref, bo_ref,
                 o_ref, ctx_ref, *, nh, dh):
    T = k_ref.shape[0]
    scores = i_ref[...]
    s = _monotone_i32(scores)
    thr = thr_ref[:, 0:1]
    m = m_ref[:, 0:1]
    eq = (s == thr).astype(jnp.int32)
    r = eq
    sh = 1
    while sh < T:
        r = r + jnp.concatenate(
            [jnp.zeros((BLK_T, sh), jnp.int32), r[:, :-sh]], axis=1)
        sh *= 2
    sel = (s > thr) | ((eq == 1) & (r <= m))
    sel = sel & (scores > NEG / 2)

    i_masked = jnp.where(sel, scores, NEG)
    mi = jnp.max(i_masked, axis=1, keepdims=True)
    e = jnp.where(sel, jnp.exp(scores - mi), 0.0)
    zi = jnp.sum(e, axis=1, keepdims=True)
    rw = e / zi

    scale = 1.0 / (dh ** 0.5)
    for h in range(nh):
        qh = q_ref[:, h * dh:(h + 1) * dh]
        kh = k_ref[:, h * dh:(h + 1) * dh]
        logits = jax.lax.dot_general(
            qh, kh, (((1,), (1,)), ((), ())),
            preferred_element_type=jnp.float32) * scale
        lm = jnp.where(sel, logits, NEG)
        ml = jnp.max(lm, axis=1, keepdims=True)
        p = jnp.where(sel, jnp.exp(logits - ml), 0.0)
        zl = jnp.sum(p, axis=1, keepdims=True)
        pw = p * rw / zl
        ctx_ref[:, h * dh:(h + 1) * dh] = jnp.dot(
            pw, v_ref[:, h * dh:(h + 1) * dh],
            preferred_element_type=jnp.float32)
    o_ref[...] = (
        jnp.dot(ctx_ref[...], wo_ref[...], preferred_element_type=jnp.float32)
        + bo_ref[...]
    )


def _attn_call(iscores, thr, m, q2d, k2d, v2d, wout, bout2d, nh, dh):
    T, D = q2d.shape
    grid = (T // BLK_T,)
    kern = functools.partial(_attn_kernel, nh=nh, dh=dh)
    return pl.pallas_call(
        kern,
        grid=grid,
        in_specs=[
            pl.BlockSpec((BLK_T, T), lambda i: (i, 0)),
            pl.BlockSpec((BLK_T, 128), lambda i: (i, 0)),
            pl.BlockSpec((BLK_T, 128), lambda i: (i, 0)),
            pl.BlockSpec((BLK_T, D), lambda i: (i, 0)),
            pl.BlockSpec((T, D), lambda i: (0, 0)),
            pl.BlockSpec((T, D), lambda i: (0, 0)),
            pl.BlockSpec((D, D), lambda i: (0, 0)),
            pl.BlockSpec((1, D), lambda i: (0, 0)),
        ],
        out_specs=pl.BlockSpec((BLK_T, D), lambda i: (i, 0)),
        out_shape=jax.ShapeDtypeStruct((T, D), jnp.float32),
        scratch_shapes=[pltpu.VMEM((BLK_T, D), jnp.float32)],
    )(iscores, thr, m, q2d, k2d, v2d, wout, bout2d)


def kernel(x, w_ih, Wq_idx, bq_idx, Wk_idx, bk_idx, Wqkv, bqkv, Wout, bout):
    B, T, D = x.shape
    HIDI = Wq_idx.shape[1]
    hi = w_ih.shape[0]
    di = HIDI // hi
    dh = 64
    nh = D // dh

    x2d = x.reshape(T, D)
    wcat = jnp.concatenate([Wqkv, Wq_idx, Wk_idx], axis=1)
    bcat = jnp.concatenate([bqkv, bq_idx, bk_idx], axis=0).reshape(1, -1)

    proj = _proj_call(x2d, wcat, bcat)
    q2d = proj[:, 0:D]
    k2d = proj[:, D:2 * D]
    v2d = proj[:, 2 * D:3 * D]
    qi = proj[:, 3 * D:3 * D + HIDI]
    ki = proj[:, 3 * D + HIDI:3 * D + 2 * HIDI]

    iscores, thr, m = _scores_call(qi, ki, w_ih.reshape(1, hi), hi, di)
    y = _attn_call(iscores, thr, m, q2d, k2d, v2d, Wout,
                   bout.reshape(1, D), nh, dh)
    return y.reshape(B, T, D)
